# baseline (speedup 1.0000x reference)
import functools

import jax
import jax.numpy as jnp
import numpy as np
from jax.experimental import pallas as pl
from jax.experimental.pallas import tpu as pltpu

_N_CLASSES = 128
_N_BINS = 10
_BOUNDS = [np.float32(v) for v in np.linspace(0.0, 1.0, _N_BINS + 1)[:-1]]

_ROWS = 4096
_CORES = 2


def _hist_kernel(x_ref, t_ref, part_ref, acc_ref, *, n_inner):
    step = pl.program_id(1)

    x = x_ref[...]
    t = t_ref[...]
    m = jnp.max(x, axis=1, keepdims=True)
    e = jnp.exp(x - m)
    s = jnp.sum(e, axis=1, keepdims=True)
    r = 1.0 / s
    p = e * r

    cls = jax.lax.broadcasted_iota(jnp.int32, (_ROWS, _N_CLASSES), 1)
    gt = t == cls
    q = jnp.where(gt, 1.0 - p, -p)

    rows = []
    rows.append(jnp.sum(q, axis=0, keepdims=True))
    for u in _BOUNDS[1:]:
        sel = jnp.where(p > u, q, 0.0)
        rows.append(jnp.sum(sel, axis=0, keepdims=True))
    upd = jnp.concatenate(
        rows + [jnp.zeros((16 - _N_BINS, _N_CLASSES), jnp.float32)], axis=0)

    @pl.when(step == 0)
    def _():
        acc_ref[...] = upd

    @pl.when(step > 0)
    def _():
        acc_ref[...] = acc_ref[...] + upd

    @pl.when(step == n_inner - 1)
    def _():
        part_ref[0] = acc_ref[...]


def _finalize_kernel(part_ref, loss_ref, *, total):
    a = part_ref[0] + part_ref[1]
    d_cum = a[0:_N_BINS]
    d_next = jnp.concatenate(
        [a[1:_N_BINS], jnp.zeros((1, _N_CLASSES), jnp.float32)], axis=0)
    per_bin = d_cum - d_next
    loss_ref[0, 0] = jnp.sum(jnp.abs(per_bin)) / total


def kernel(output, target):
    batch, n_classes = output.shape
    n_inner = batch // (_ROWS * _CORES)
    t2 = target.reshape(batch, 1)

    parts = pl.pallas_call(
        functools.partial(_hist_kernel, n_inner=n_inner),
        grid=(_CORES, n_inner),
        in_specs=[
            pl.BlockSpec((_ROWS, n_classes), lambda i, j: (i * n_inner + j, 0)),
            pl.BlockSpec((_ROWS, 1), lambda i, j: (i * n_inner + j, 0)),
        ],
        out_specs=pl.BlockSpec((1, 16, n_classes), lambda i, j: (i, 0, 0)),
        out_shape=jax.ShapeDtypeStruct((_CORES, 16, n_classes), jnp.float32),
        scratch_shapes=[pltpu.VMEM((16, _N_CLASSES), jnp.float32)],
        compiler_params=pltpu.CompilerParams(
            dimension_semantics=("parallel", "arbitrary")),
    )(output, t2)

    loss = pl.pallas_call(
        functools.partial(_finalize_kernel, total=float(batch * n_classes)),
        out_specs=pl.BlockSpec(memory_space=pltpu.SMEM),
        out_shape=jax.ShapeDtypeStruct((1, 1), jnp.float32),
    )(parts)
    return loss[0, 0]
